# baseline (speedup 1.0000x reference)
import functools

import jax
import jax.numpy as jnp
from jax import lax
from jax.experimental import pallas as pl
from jax.experimental.pallas import tpu as pltpu
from jax.experimental.pallas import tpu_sc as plsc

BATCH = 4096
INPUT_DIM = 39
N_CONT = 13
N_CAT = 26
VOCAB = 100000
EMB = 32
LANES = 16
OUT_DIM = N_CONT + N_CAT * EMB

NUM_CORES = 2
NUM_SUBCORES = 16
NUM_WORKERS = NUM_CORES * NUM_SUBCORES
B_PER_W = BATCH // NUM_WORKERS
VREGS_PER_TABLE = B_PER_W // LANES

_mesh = plsc.VectorSubcoreMesh(
    core_axis_name="c", subcore_axis_name="s",
    num_cores=NUM_CORES, num_subcores=NUM_SUBCORES,
)


@functools.partial(
    pl.kernel,
    out_type=jax.ShapeDtypeStruct((OUT_DIM, BATCH), jnp.float32),
    mesh=_mesh,
    compiler_params=pltpu.CompilerParams(
        use_tc_tiling_on_sc=False, needs_layout_passes=False),
    scratch_types=[
        pltpu.VMEM((N_CAT, B_PER_W), jnp.int32),
        pltpu.VMEM((OUT_DIM, B_PER_W), jnp.float32),
        pltpu.VMEM((B_PER_W, EMB), jnp.float32),
        pltpu.SemaphoreType.DMA,
    ],
)
def _emb_kernel(tables_hbm, gidx_hbm, xt_hbm, out_hbm, gidx_v, asm_v, buf_v, sem):
    wid = lax.axis_index("s") * NUM_CORES + lax.axis_index("c")
    base_b = wid * B_PER_W

    pltpu.sync_copy(gidx_hbm.at[:, pl.ds(base_b, B_PER_W)], gidx_v)

    pltpu.sync_copy(xt_hbm.at[pl.ds(0, N_CONT), pl.ds(base_b, B_PER_W)],
                    asm_v.at[pl.ds(0, N_CONT), :])

    lanes_iota = lax.iota(jnp.int32, LANES)

    def per_table(j, carry):
        gathers = []
        for h in range(VREGS_PER_TABLE):
            idx16 = gidx_v[j, pl.ds(h * LANES, LANES)]
            gathers.append(pltpu.async_copy(
                tables_hbm.at[idx16],
                buf_v.at[pl.ds(h * LANES, LANES), :],
                sem))
        for g in gathers:
            g.wait()
        row0 = N_CONT + j * EMB
        for e in range(EMB):
            col_idx = jnp.full((LANES,), e, jnp.int32)
            for h in range(VREGS_PER_TABLE):
                vals = plsc.load_gather(
                    buf_v, [h * LANES + lanes_iota, col_idx])
                asm_v[row0 + e, pl.ds(h * LANES, LANES)] = vals
        return carry

    lax.fori_loop(0, N_CAT, per_table, 0)

    pltpu.sync_copy(asm_v, out_hbm.at[:, pl.ds(base_b, B_PER_W)])


def kernel(x, tables):
    xt = x.T
    gidx = xt[N_CONT:].astype(jnp.int32) + (
        jnp.arange(N_CAT, dtype=jnp.int32) * VOCAB
    )[:, None]
    tables_flat = tables.reshape(N_CAT * VOCAB, EMB)
    out_t = _emb_kernel(tables_flat, gidx, xt)
    return out_t.T


# speedup vs baseline: 1.0203x; 1.0203x over previous
import functools

import jax
import jax.numpy as jnp
from jax import lax
from jax.experimental import pallas as pl
from jax.experimental.pallas import tpu as pltpu
from jax.experimental.pallas import tpu_sc as plsc

BATCH = 4096
INPUT_DIM = 39
N_CONT = 13
N_CAT = 26
VOCAB = 100000
EMB = 32
LANES = 16
OUT_DIM = N_CONT + N_CAT * EMB

NUM_CORES = 2
NUM_SUBCORES = 16
NUM_WORKERS = NUM_CORES * NUM_SUBCORES
B_PER_W = BATCH // NUM_WORKERS
NPASS = 2
B_INNER = B_PER_W // NPASS
VREGS_PER_TABLE = B_INNER // LANES

_mesh = plsc.VectorSubcoreMesh(
    core_axis_name="c", subcore_axis_name="s",
    num_cores=NUM_CORES, num_subcores=NUM_SUBCORES,
)


@functools.partial(
    pl.kernel,
    out_type=jax.ShapeDtypeStruct((OUT_DIM, BATCH), jnp.float32),
    mesh=_mesh,
    compiler_params=pltpu.CompilerParams(
        use_tc_tiling_on_sc=False, needs_layout_passes=False),
    scratch_types=[
        pltpu.VMEM((N_CAT, B_PER_W), jnp.float32),
        pltpu.VMEM((OUT_DIM, B_INNER), jnp.float32),
        pltpu.VMEM((B_INNER, EMB), jnp.float32),
        pltpu.SemaphoreType.DMA,
    ],
)
def _emb_kernel(tables_hbm, xt_hbm, out_hbm, xi_v, asm_v, buf_v, sem):
    wid = lax.axis_index("s") * NUM_CORES + lax.axis_index("c")
    base_b = wid * B_PER_W

    pltpu.sync_copy(
        xt_hbm.at[pl.ds(N_CONT, N_CAT), pl.ds(base_b, B_PER_W)], xi_v)

    iota = lax.iota(jnp.int32, LANES)

    for p in range(NPASS):
        pass_b = base_b + p * B_INNER

        pltpu.sync_copy(
            xt_hbm.at[pl.ds(0, N_CONT), pl.ds(pass_b, B_INNER)],
            asm_v.at[pl.ds(0, N_CONT), :])

        def per_table(j, carry, p=p):
            off = jnp.full((LANES,), 0, jnp.int32) + j * VOCAB
            gathers = []
            for h in range(VREGS_PER_TABLE):
                lane0 = p * B_INNER + h * LANES
                idx16 = xi_v[j, pl.ds(lane0, LANES)].astype(jnp.int32) + off
                gathers.append(pltpu.async_copy(
                    tables_hbm.at[idx16],
                    buf_v.at[pl.ds(h * LANES, LANES), :],
                    sem))
            for g in gathers:
                g.wait()
            row0 = N_CONT + j * EMB
            for e in range(EMB):
                rot_e = (e + iota) % EMB
                dst_rows = row0 + rot_e
                for h in range(VREGS_PER_TABLE):
                    vals = plsc.load_gather(buf_v, [h * LANES + iota, rot_e])
                    plsc.store_scatter(
                        asm_v, [dst_rows, h * LANES + iota], vals)
            return carry

        lax.fori_loop(0, N_CAT, per_table, 0)

        pltpu.sync_copy(asm_v, out_hbm.at[:, pl.ds(pass_b, B_INNER)])


def kernel(x, tables):
    xt = x.T
    tables_flat = tables.reshape(N_CAT * VOCAB, EMB)
    out_t = _emb_kernel(tables_flat, xt)
    return out_t.T


# speedup vs baseline: 1.9586x; 1.9197x over previous
import functools

import jax
import jax.numpy as jnp
from jax import lax
from jax.experimental import pallas as pl
from jax.experimental.pallas import tpu as pltpu
from jax.experimental.pallas import tpu_sc as plsc

BATCH = 4096
INPUT_DIM = 39
N_CONT = 13
N_CAT = 26
VOCAB = 100000
EMB = 32
LANES = 16
OUT_DIM = N_CONT + N_CAT * EMB

NUM_CORES = 2
NUM_SUBCORES = 16
NUM_WORKERS = NUM_CORES * NUM_SUBCORES
B_PER_W = BATCH // NUM_WORKERS
VREGS_PER_TABLE = B_PER_W // LANES

_mesh = plsc.VectorSubcoreMesh(
    core_axis_name="c", subcore_axis_name="s",
    num_cores=NUM_CORES, num_subcores=NUM_SUBCORES,
)


@functools.partial(
    pl.kernel,
    out_type=jax.ShapeDtypeStruct((OUT_DIM, BATCH), jnp.float32),
    mesh=_mesh,
    compiler_params=pltpu.CompilerParams(
        use_tc_tiling_on_sc=False, needs_layout_passes=False),
    scratch_types=[
        pltpu.VMEM((N_CAT, B_PER_W), jnp.float32),
        pltpu.VMEM((OUT_DIM, B_PER_W), jnp.float32),
        pltpu.SemaphoreType.DMA,
    ],
)
def _emb_kernel(tt_hbm, xt_hbm, out_hbm, xi_v, asm_v, sem):
    wid = lax.axis_index("s") * NUM_CORES + lax.axis_index("c")
    base_b = wid * B_PER_W

    pltpu.sync_copy(
        xt_hbm.at[pl.ds(N_CONT, N_CAT), pl.ds(base_b, B_PER_W)], xi_v)
    pltpu.sync_copy(xt_hbm.at[pl.ds(0, N_CONT), pl.ds(base_b, B_PER_W)],
                    asm_v.at[pl.ds(0, N_CONT), :])

    def per_table(j, carry):
        vjs = [
            xi_v[j, pl.ds(h * LANES, LANES)].astype(jnp.int32)
            for h in range(VREGS_PER_TABLE)
        ]
        row0 = N_CONT + j * EMB
        copies = []
        for e in range(EMB):
            base = (j * EMB + e) * VOCAB
            for h in range(VREGS_PER_TABLE):
                flat_idx = vjs[h] + base
                copies.append(pltpu.async_copy(
                    tt_hbm.at[flat_idx],
                    asm_v.at[row0 + e, pl.ds(h * LANES, LANES)],
                    sem))
        for c in copies:
            c.wait()
        return carry

    lax.fori_loop(0, N_CAT, per_table, 0)

    pltpu.sync_copy(asm_v, out_hbm.at[:, pl.ds(base_b, B_PER_W)])


def kernel(x, tables):
    xt = x.T
    tt = jnp.transpose(tables, (0, 2, 1)).reshape(N_CAT * EMB * VOCAB)
    out_t = _emb_kernel(tt, xt)
    return out_t.T


# speedup vs baseline: 2.0205x; 1.0316x over previous
import functools

import jax
import jax.numpy as jnp
from jax import lax
from jax.experimental import pallas as pl
from jax.experimental.pallas import tpu as pltpu
from jax.experimental.pallas import tpu_sc as plsc

BATCH = 4096
INPUT_DIM = 39
N_CONT = 13
N_CAT = 26
VOCAB = 100000
EMB = 32
LANES = 16
OUT_DIM = N_CONT + N_CAT * EMB

NUM_CORES = 2
NUM_SUBCORES = 16
NUM_WORKERS = NUM_CORES * NUM_SUBCORES
B_PER_W = BATCH // NUM_WORKERS
VREGS_PER_TABLE = B_PER_W // LANES

_mesh = plsc.VectorSubcoreMesh(
    core_axis_name="c", subcore_axis_name="s",
    num_cores=NUM_CORES, num_subcores=NUM_SUBCORES,
)


@functools.partial(
    pl.kernel,
    out_type=jax.ShapeDtypeStruct((OUT_DIM, BATCH), jnp.float32),
    mesh=_mesh,
    compiler_params=pltpu.CompilerParams(
        use_tc_tiling_on_sc=False, needs_layout_passes=False),
    scratch_types=[
        pltpu.VMEM((N_CAT, B_PER_W), jnp.float32),
        pltpu.VMEM((OUT_DIM, B_PER_W), jnp.float32),
        pltpu.SemaphoreType.DMA,
    ],
)
def _emb_kernel(tt_hbm, xt_hbm, out_hbm, xi_v, asm_v, sem):
    wid = lax.axis_index("s") * NUM_CORES + lax.axis_index("c")
    base_b = wid * B_PER_W

    pltpu.sync_copy(
        xt_hbm.at[pl.ds(N_CONT, N_CAT), pl.ds(base_b, B_PER_W)], xi_v)
    pltpu.sync_copy(xt_hbm.at[pl.ds(0, N_CONT), pl.ds(base_b, B_PER_W)],
                    asm_v.at[pl.ds(0, N_CONT), :])

    def table_bytes_wait():
        pltpu.make_async_copy(
            xt_hbm.at[pl.ds(0, EMB), pl.ds(0, B_PER_W)],
            asm_v.at[pl.ds(N_CONT, EMB), :],
            sem).wait()

    def per_table(j, carry):
        vjs = [
            xi_v[j, pl.ds(h * LANES, LANES)].astype(jnp.int32)
            for h in range(VREGS_PER_TABLE)
        ]
        row0 = N_CONT + j * EMB
        for e in range(EMB):
            base = (j * EMB + e) * VOCAB
            for h in range(VREGS_PER_TABLE):
                flat_idx = vjs[h] + base
                pltpu.async_copy(
                    tt_hbm.at[flat_idx],
                    asm_v.at[row0 + e, pl.ds(h * LANES, LANES)],
                    sem)

        @pl.when(j > 0)
        def _():
            table_bytes_wait()

        return carry

    lax.fori_loop(0, N_CAT, per_table, 0)
    table_bytes_wait()

    pltpu.sync_copy(asm_v, out_hbm.at[:, pl.ds(base_b, B_PER_W)])


def kernel(x, tables):
    xt = x.T
    tt = jnp.transpose(tables, (0, 2, 1)).reshape(N_CAT * EMB * VOCAB)
    out_t = _emb_kernel(tt, xt)
    return out_t.T
